# baseline (speedup 1.0000x reference)
import functools
import jax
import jax.numpy as jnp
from jax import lax
from jax.experimental import pallas as pl
from jax.experimental.pallas import tpu as pltpu
from jax.experimental.pallas import tpu_sc as plsc

_INFO = plsc.get_sparse_core_info()
_NC = _INFO.num_cores
_NS = _INFO.num_subcores
_NW = _NC * _NS


def _gather_call(table, index):
    B, = index.shape
    V, D = table.shape
    assert B % _NW == 0
    b_per_w = B // _NW
    C = 1664
    assert b_per_w % C == 0
    n_chunks = b_per_w // C

    mesh = plsc.VectorSubcoreMesh(core_axis_name="c", subcore_axis_name="s")

    @functools.partial(
        pl.kernel,
        mesh=mesh,
        out_type=jax.ShapeDtypeStruct((B, D), table.dtype),
        scratch_types=[
            pltpu.VMEM((C,), jnp.int32),
            pltpu.VMEM((C, D), table.dtype),
            pltpu.SemaphoreType.DMA,
        ],
        compiler_params=pltpu.CompilerParams(use_tc_tiling_on_sc=False),
    )
    def k(table_hbm, idx_hbm, out_hbm, idx_v, rows_v, sem):
        wid = lax.axis_index("s") * _NC + lax.axis_index("c")
        base = wid * b_per_w

        def chunk(i, carry):
            off = base + i * C
            pltpu.sync_copy(idx_hbm.at[pl.ds(off, C)], idx_v)
            pltpu.async_copy(table_hbm.at[idx_v], rows_v, sem).wait()
            pltpu.sync_copy(rows_v, out_hbm.at[pl.ds(off, C)])
            return carry

        lax.fori_loop(0, n_chunks, chunk, 0)

    return k(table, index)


def kernel(input, index, _):
    out = _gather_call(input, index)
    return (input, index, out)


# speedup vs baseline: 1.0070x; 1.0070x over previous
import functools
import jax
import jax.numpy as jnp
from jax import lax
from jax.experimental import pallas as pl
from jax.experimental.pallas import tpu as pltpu
from jax.experimental.pallas import tpu_sc as plsc

_INFO = plsc.get_sparse_core_info()
_NC = _INFO.num_cores
_NS = _INFO.num_subcores
_NW = _NC * _NS


def _gather_call(table, index):
    B, = index.shape
    V, D = table.shape
    assert B % _NW == 0
    b_per_w = B // _NW
    C = 1664
    NBUF = 2
    assert b_per_w % C == 0
    n_chunks = b_per_w // C

    mesh = plsc.VectorSubcoreMesh(core_axis_name="c", subcore_axis_name="s")

    @functools.partial(
        pl.kernel,
        mesh=mesh,
        out_type=jax.ShapeDtypeStruct((B, D), table.dtype),
        scratch_types=[
            pltpu.VMEM((b_per_w,), jnp.int32),
            [pltpu.VMEM((C, D), table.dtype) for _ in range(NBUF)],
            [pltpu.SemaphoreType.DMA for _ in range(NBUF)],
            [pltpu.SemaphoreType.DMA for _ in range(NBUF)],
        ],
        compiler_params=pltpu.CompilerParams(use_tc_tiling_on_sc=False),
    )
    def k(table_hbm, idx_hbm, out_hbm, idx_v, rows, g_sems, w_sems):
        wid = lax.axis_index("s") * _NC + lax.axis_index("c")
        base = wid * b_per_w
        pltpu.sync_copy(idx_hbm.at[pl.ds(base, b_per_w)], idx_v)

        def start_gather(i):
            b = i % NBUF
            pltpu.async_copy(
                table_hbm.at[idx_v.at[pl.ds(i * C, C)]], rows[b], g_sems[b]
            )

        def wait_gather(i):
            b = i % NBUF
            pltpu.make_async_copy(
                table_hbm.at[idx_v.at[pl.ds(i * C, C)]], rows[b], g_sems[b]
            ).wait()

        def start_write(i):
            b = i % NBUF
            pltpu.async_copy(rows[b], out_hbm.at[pl.ds(base + i * C, C)], w_sems[b])

        def wait_write(i):
            b = i % NBUF
            pltpu.make_async_copy(
                rows[b], out_hbm.at[pl.ds(base + i * C, C)], w_sems[b]
            ).wait()

        for i in range(min(NBUF, n_chunks)):
            start_gather(i)
        for i in range(n_chunks):
            wait_gather(i)
            start_write(i)
            nxt = i + NBUF
            if nxt < n_chunks:
                wait_write(nxt - NBUF)
                start_gather(nxt)
        for i in range(max(0, n_chunks - NBUF), n_chunks):
            wait_write(i)

    return k(table, index)


def kernel(input, index, _):
    out = _gather_call(input, index)
    return (input, index, out)
